# Initial kernel scaffold; baseline (speedup 1.0000x reference)
import jax
import jax.numpy as jnp
from jax.experimental import pallas as pl


def kernel(x, edge_index, edge_attr, batch_index, params):
    raise NotImplementedError("write your pallas kernel here")


# speedup vs baseline: 2.3998x; 2.3998x over previous
import functools

import jax
import jax.numpy as jnp
from jax import lax
from jax.experimental import pallas as pl
from jax.experimental.pallas import tpu as pltpu
from jax.experimental.pallas import tpu_sc as plsc

F32 = jnp.float32
EMB = 64
N_NODES = 10000
N_EDGES = 320000
NUM_GRAPHS = 64
CHUNK = 128
N_CHUNKS = N_EDGES // CHUNK
NSC = 2
NW = 16 * NSC
PH_ROWS = 3336
DUMMY = PH_ROWS
ACC_ROWS = PH_ROWS + 8
ROWS_MAIN = 208
EDGE_BLK = 8000


def _mm_ln_body(x_ref, w_ref, b_ref, g_ref, bt_ref, o_ref):
    y = jnp.dot(x_ref[...], w_ref[...], preferred_element_type=F32) + b_ref[...]
    mu = jnp.mean(y, axis=1, keepdims=True)
    var = jnp.mean((y - mu) ** 2, axis=1, keepdims=True)
    o_ref[...] = (y - mu) / jnp.sqrt(var + 1e-5) * g_ref[...] + bt_ref[...]


def _lt_ln(x, lt, bn):
    W, b = lt
    g, bt = bn
    n = x.shape[0]
    return pl.pallas_call(
        _mm_ln_body,
        out_shape=jax.ShapeDtypeStruct((n, W.shape[1]), F32),
    )(x, W, b.reshape(1, -1), g.reshape(1, -1), bt.reshape(1, -1))


def _mm_body(x_ref, w_ref, b_ref, o_ref):
    o_ref[...] = jnp.dot(x_ref[...], w_ref[...], preferred_element_type=F32) + b_ref[...]


def _edge_linear(e, lt):
    W, b = lt
    k, dout = W.shape
    grid = N_EDGES // EDGE_BLK
    return pl.pallas_call(
        _mm_body,
        grid=(grid,),
        in_specs=[
            pl.BlockSpec((EDGE_BLK, k), lambda i: (i, 0)),
            pl.BlockSpec((k, dout), lambda i: (0, 0)),
            pl.BlockSpec((1, dout), lambda i: (0, 0)),
        ],
        out_specs=pl.BlockSpec((EDGE_BLK, dout), lambda i: (i, 0)),
        out_shape=jax.ShapeDtypeStruct((N_EDGES, dout), F32),
    )(e, W, b.reshape(1, -1))


def _node3_body(x_ref, w_ref, b_ref, ax_ref, dx_ref, eb_ref):
    y = jnp.dot(x_ref[...], w_ref[...], preferred_element_type=F32) + b_ref[...]
    ax_ref[...] = y[:, :EMB]
    dx_ref[:, :EMB] = y[:, EMB:2 * EMB]
    dx_ref[:, EMB:] = jnp.zeros((N_NODES, EMB), F32)
    eb_ref[...] = y[:, 2 * EMB:]


def _node3(x, W, b):
    return pl.pallas_call(
        _node3_body,
        out_shape=[
            jax.ShapeDtypeStruct((N_NODES, EMB), F32),
            jax.ShapeDtypeStruct((N_NODES, 2 * EMB), F32),
            jax.ShapeDtypeStruct((N_NODES, 2 * EMB), F32),
        ],
    )(x, W, b.reshape(1, -1))


def _post_node_body(acc_ref, ax_ref, xin_ref, g2_ref, b2_ref, w2_ref, bw2_ref,
                    g3_ref, b3_ref, o_ref):
    t = acc_ref[0] + acc_ref[1]
    seg0 = t[:PH_ROWS]
    seg1 = t[PH_ROWS:2 * PH_ROWS] - t[:PH_ROWS]
    seg2 = t[2 * PH_ROWS:] - t[PH_ROWS:PH_ROWS + (N_NODES - 2 * PH_ROWS)]
    acc_full = jnp.concatenate([seg0, seg1, seg2], axis=0)
    ssx = acc_full[:, :EMB]
    ss = acc_full[:, EMB:]
    xn = ax_ref[...] + ssx / (ss + 1e-6)
    mu = jnp.mean(xn, axis=0, keepdims=True)
    var = jnp.mean((xn - mu) ** 2, axis=0, keepdims=True)
    xn = jnp.maximum((xn - mu) / jnp.sqrt(var + 1e-5), 0.0)
    x1 = xin_ref[...] + xn
    mu = jnp.mean(x1, axis=1, keepdims=True)
    var = jnp.mean((x1 - mu) ** 2, axis=1, keepdims=True)
    x1 = (x1 - mu) / jnp.sqrt(var + 1e-5) * g2_ref[...] + b2_ref[...]
    y = jnp.dot(x1, w2_ref[...], preferred_element_type=F32) + bw2_ref[...]
    mu = jnp.mean(y, axis=1, keepdims=True)
    var = jnp.mean((y - mu) ** 2, axis=1, keepdims=True)
    o_ref[...] = (y - mu) / jnp.sqrt(var + 1e-5) * g3_ref[...] + b3_ref[...]


def _post_node(acc, ax, xin, bn2, lt2, bn3):
    return pl.pallas_call(
        _post_node_body,
        out_shape=jax.ShapeDtypeStruct((N_NODES, EMB), F32),
    )(acc, ax, xin,
      bn2[0].reshape(1, -1), bn2[1].reshape(1, -1),
      lt2[0], lt2[1].reshape(1, -1),
      bn3[0].reshape(1, -1), bn3[1].reshape(1, -1))


def _pool_body(x_ref, b_ref, o_ref):
    xv = x_ref[...]
    bv = b_ref[...]
    rows = lax.broadcasted_iota(jnp.int32, (NUM_GRAPHS, 2 * EMB), 0)

    def g_body(g, out):
        m = bv == g
        cnt = jnp.sum(m.astype(F32))
        s = jnp.sum(jnp.where(m, xv, 0.0), axis=0)
        mx = jnp.max(jnp.where(m, xv, -jnp.inf), axis=0)
        row = jnp.concatenate([mx, s / jnp.maximum(cnt, 1.0)])
        return jnp.where(rows == g, row[None, :], out)

    o_ref[...] = lax.fori_loop(0, NUM_GRAPHS, g_body, jnp.zeros((NUM_GRAPHS, 2 * EMB), F32))


def _pool(xp, batch2d):
    return pl.pallas_call(
        _pool_body,
        out_shape=jax.ShapeDtypeStruct((NUM_GRAPHS, 2 * EMB), F32),
    )(xp, batch2d)


def _edge_bn_body(eij_ref, e_ref, st_ref, o_ref):
    st = st_ref[...]
    s = jnp.sum(st[:, :EMB], axis=0, keepdims=True)
    q = jnp.sum(st[:, EMB:], axis=0, keepdims=True)
    mu = s / N_EDGES
    var = q / N_EDGES - mu * mu
    o_ref[...] = e_ref[...] + jnp.maximum((eij_ref[...] - mu) / jnp.sqrt(var + 1e-5), 0.0)


def _edge_bn_apply(eij, e, stats):
    grid = N_EDGES // EDGE_BLK
    return pl.pallas_call(
        _edge_bn_body,
        grid=(grid,),
        in_specs=[
            pl.BlockSpec((EDGE_BLK, EMB), lambda i: (i, 0)),
            pl.BlockSpec((EDGE_BLK, EMB), lambda i: (i, 0)),
            pl.BlockSpec((NW, 2 * EMB), lambda i: (0, 0)),
        ],
        out_specs=pl.BlockSpec((EDGE_BLK, EMB), lambda i: (i, 0)),
        out_shape=jax.ShapeDtypeStruct((N_EDGES, EMB), F32),
    )(eij, e, stats)


def _classifier_body(x1_ref, x2_ref, x3_ref, w1_ref, b1_ref, g_ref, bt_ref,
                     w2_ref, b2_ref, o_ref):
    h = x1_ref[...] + x2_ref[...] + x3_ref[...]
    h = jnp.maximum(jnp.dot(h, w1_ref[...], preferred_element_type=F32) + b1_ref[...], 0.0)
    mu = jnp.mean(h, axis=1, keepdims=True)
    var = jnp.mean((h - mu) ** 2, axis=1, keepdims=True)
    h = (h - mu) / jnp.sqrt(var + 1e-5) * g_ref[...] + bt_ref[...]
    o_ref[...] = jnp.dot(h, w2_ref[...], preferred_element_type=F32) + b2_ref[...]


def _classifier(x1, x2, x3, c1, bnl, c2):
    return pl.pallas_call(
        _classifier_body,
        out_shape=jax.ShapeDtypeStruct((NUM_GRAPHS, c2[0].shape[1]), F32),
    )(x1, x2, x3, c1[0], c1[1].reshape(1, -1),
      bnl[0].reshape(1, -1), bnl[1].reshape(1, -1),
      c2[0], c2[1].reshape(1, -1))


def _edge_pass(dxT, ebT, ce, src, dst):
    mesh = plsc.VectorSubcoreMesh(core_axis_name="c", subcore_axis_name="s", num_cores=NSC)

    @functools.partial(
        pl.kernel,
        out_type=[
            jax.ShapeDtypeStruct((N_EDGES, EMB), F32),
            jax.ShapeDtypeStruct((NSC, N_NODES, 2 * EMB), F32),
            jax.ShapeDtypeStruct((NW, 1, 2 * EMB), F32),
            jax.ShapeDtypeStruct((N_EDGES, 2 * EMB), F32),
        ],
        scratch_types=[
            pltpu.VMEM((CHUNK,), jnp.int32),
            pltpu.VMEM((CHUNK,), jnp.int32),
            pltpu.VMEM((CHUNK,), jnp.int32),
            pltpu.VMEM((CHUNK, 2 * EMB), F32),
            pltpu.VMEM((CHUNK, 2 * EMB), F32),
            pltpu.VMEM((CHUNK, EMB), F32),
            pltpu.VMEM((CHUNK, EMB), F32),
            pltpu.VMEM((CHUNK, 2 * EMB), F32),
            pltpu.VMEM((1, 2 * EMB), F32),
            pltpu.VMEM((104, 2 * EMB), F32),
            pltpu.VMEM_SHARED((ACC_ROWS, 2 * EMB), F32),
            pltpu.SemaphoreType.DMA,
            pltpu.SemaphoreType.DMA,
        ],
        mesh=mesh,
    )
    def edge_kernel(dx_hbm, eb_hbm, ce_hbm, src_hbm, dst_hbm,
                    eij_o, acc_o, st_o, sv_o,
                    src_v, dst_v, idx_v, dxv, ebv, cev, eijv, scatv, statv,
                    zbuf, acc_sh, sem1, sem2):
        cid = lax.axis_index("c")
        sid = lax.axis_index("s")
        wid = sid * NSC + cid

        zv = jnp.zeros((16,), F32)

        def zb_body(r, carry):
            for c in range(8):
                zbuf[r, pl.ds(c * 16, 16)] = zv
            return carry

        lax.fori_loop(0, 104, zb_body, 0)

        for k in range(2):
            pltpu.sync_copy(zbuf.at[pl.ds(0, 104)],
                            acc_sh.at[pl.ds(sid * ROWS_MAIN + k * 104, 104)])

        @pl.when(sid == 15)
        def _zero_tail():
            pltpu.sync_copy(zbuf.at[pl.ds(0, 16)],
                            acc_sh.at[pl.ds(16 * ROWS_MAIN, 16)])

        def copy_out(node_base, nrows):
            pltpu.sync_copy(acc_sh.at[pl.ds(sid * ROWS_MAIN, ROWS_MAIN)],
                            acc_o.at[cid, pl.ds(node_base + sid * ROWS_MAIN, ROWS_MAIN)])
            tail = nrows - 16 * ROWS_MAIN
            if tail:
                @pl.when(sid == 15)
                def _copy_tail():
                    pltpu.sync_copy(
                        acc_sh.at[pl.ds(16 * ROWS_MAIN, tail)],
                        acc_o.at[cid, pl.ds(node_base + 16 * ROWS_MAIN, tail)])

        plsc.subcore_barrier()

        n_chunks = jnp.where(wid < N_CHUNKS - (N_CHUNKS // NW) * NW,
                             N_CHUNKS // NW + 1, N_CHUNKS // NW)

        def chunk_body(j, st):
            chunk = wid + NW * j
            base = chunk * CHUNK
            pltpu.sync_copy(src_hbm.at[pl.ds(base, CHUNK)], src_v)
            pltpu.sync_copy(dst_hbm.at[pl.ds(base, CHUNK)], dst_v)
            g1 = pltpu.async_copy(dx_hbm.at[dst_v], dxv, sem1)
            g2 = pltpu.async_copy(eb_hbm.at[src_v], ebv, sem2)
            pltpu.sync_copy(ce_hbm.at[pl.ds(base, CHUNK)], cev)
            g1.wait()
            g2.wait()

            def row_body(r, st_in):
                out = list(st_in)
                for c in range(4):
                    sl = pl.ds(c * 16, 16)
                    slb = pl.ds(EMB + c * 16, 16)
                    eij = dxv[r, sl] + ebv[r, sl] + cev[r, sl]
                    sig = 1.0 / (1.0 + jnp.exp(-eij))
                    eijv[r, sl] = eij
                    scatv[r, sl] = sig * ebv[r, slb]
                    scatv[r, slb] = sig
                    out[c] = st_in[c] + eij
                    out[4 + c] = st_in[4 + c] + eij * eij
                return tuple(out)

            st = lax.fori_loop(0, CHUNK, row_body, st)
            for i in range(CHUNK // 16):
                d = dst_v[pl.ds(i * 16, 16)]
                idx_v[pl.ds(i * 16, 16)] = jnp.where(d < PH_ROWS, d, DUMMY)
            pltpu.sync_copy(eijv, eij_o.at[pl.ds(base, CHUNK)])
            pltpu.sync_copy(scatv, sv_o.at[pl.ds(base, CHUNK)])
            pltpu.sync_copy(scatv, acc_sh.at[idx_v], add=True)
            return st

        st0 = tuple(jnp.zeros((16,), F32) for _ in range(8))
        st = lax.fori_loop(0, n_chunks, chunk_body, st0)

        for c in range(4):
            statv[0, pl.ds(c * 16, 16)] = st[c]
            statv[0, pl.ds(EMB + c * 16, 16)] = st[4 + c]
        pltpu.sync_copy(statv, st_o.at[wid])

        plsc.subcore_barrier()
        copy_out(0, PH_ROWS)
        plsc.subcore_barrier()

        for ph in (1, 2):
            lo = ph * PH_ROWS
            hi = min((ph + 1) * PH_ROWS, N_NODES)

            def chunk_body2(j, carry, lo=lo, hi=hi):
                chunk = wid + NW * j
                base = chunk * CHUNK
                pltpu.sync_copy(dst_hbm.at[pl.ds(base, CHUNK)], dst_v)
                pltpu.sync_copy(sv_o.at[pl.ds(base, CHUNK)], scatv)
                for i in range(CHUNK // 16):
                    d = dst_v[pl.ds(i * 16, 16)]
                    idx_v[pl.ds(i * 16, 16)] = jnp.where(
                        (d >= lo) & (d < hi), d - lo, DUMMY)
                pltpu.sync_copy(scatv, acc_sh.at[idx_v], add=True)
                return carry

            lax.fori_loop(0, n_chunks, chunk_body2, 0)
            plsc.subcore_barrier()
            copy_out(lo, hi - lo)
            plsc.subcore_barrier()

    eij, acc, st, _sv = edge_kernel(dxT, ebT, ce, src, dst)
    return eij, acc, st.reshape(NW, 2 * EMB)


def kernel(x, edge_index, edge_attr, batch_index, params):
    p = params
    src = edge_index[0]
    dst = edge_index[1]
    batch2d = batch_index.reshape(N_NODES, 1)

    e0 = _edge_linear(edge_attr, p['edge_emb'])
    xx0 = _lt_ln(x, p['lt11'], p['bn11'])

    def stk(*a):
        return jnp.stack(a)

    convs = [p['conv1'], p['conv2'], p['conv3']]
    ps = {
        'w4': stk(*[jnp.concatenate([c['A'][0], c['D'][0], c['E'][0], c['B'][0]], axis=1)
                    for c in convs]),
        'b4': stk(*[jnp.concatenate([c['A'][1], c['D'][1], c['E'][1], c['B'][1]])
                    for c in convs]),
        'cw': stk(*[c['C'][0] for c in convs]),
        'cb': stk(*[c['C'][1] for c in convs]),
        'g2': stk(p['bn12'][0], p['bn22'][0], p['bn32'][0]),
        'bt2': stk(p['bn12'][1], p['bn22'][1], p['bn32'][1]),
        'w2': stk(p['lt12'][0], p['lt22'][0], p['lt32'][0]),
        'bw2': stk(p['lt12'][1], p['lt22'][1], p['lt32'][1]),
        'g3': stk(p['bn13'][0], p['bn23'][0], p['bn33'][0]),
        'bt3': stk(p['bn13'][1], p['bn23'][1], p['bn33'][1]),
        'wn': stk(p['lt21'][0], p['lt31'][0], p['lt31'][0]),
        'bwn': stk(p['lt21'][1], p['lt31'][1], p['lt31'][1]),
        'gn': stk(p['bn21'][0], p['bn31'][0], p['bn31'][0]),
        'btn': stk(p['bn21'][1], p['bn31'][1], p['bn31'][1]),
    }

    def conv_step(carry, cp):
        xx, e = carry
        ax, dxT, ebT = _node3(xx, cp['w4'], cp['b4'])
        ce = _edge_linear(e, (cp['cw'], cp['cb']))
        eij, acc, stats = _edge_pass(dxT, ebT, ce, src, dst)
        xp = _post_node(acc, ax, xx, (cp['g2'], cp['bt2']),
                        (cp['w2'], cp['bw2']), (cp['g3'], cp['bt3']))
        pool = _pool(xp, batch2d)
        e2 = _edge_bn_apply(eij, e, stats)
        xx2 = _lt_ln(xp, (cp['wn'], cp['bwn']), (cp['gn'], cp['btn']))
        return (xx2, e2), pool

    _, pools = lax.scan(conv_step, (xx0, e0), ps)

    return _classifier(pools[0], pools[1], pools[2],
                       p['class1'], p['bnl'], p['class2'])


# speedup vs baseline: 2.5644x; 1.0686x over previous
import functools

import jax
import jax.numpy as jnp
from jax import lax
from jax.experimental import pallas as pl
from jax.experimental.pallas import tpu as pltpu
from jax.experimental.pallas import tpu_sc as plsc

F32 = jnp.float32
EMB = 64
N_NODES = 10000
N_EDGES = 320000
NUM_GRAPHS = 64
CHUNK = 128
N_CHUNKS = N_EDGES // CHUNK
NSC = 2
NW = 16 * NSC
PH_ROWS = 3336
DUMMY = PH_ROWS
ACC_ROWS = PH_ROWS + 8
ROWS_MAIN = 208
EDGE_BLK = 8000


def _mm_ln_body(x_ref, w_ref, b_ref, g_ref, bt_ref, o_ref):
    y = jnp.dot(x_ref[...], w_ref[...], preferred_element_type=F32) + b_ref[...]
    mu = jnp.mean(y, axis=1, keepdims=True)
    var = jnp.mean((y - mu) ** 2, axis=1, keepdims=True)
    o_ref[...] = (y - mu) / jnp.sqrt(var + 1e-5) * g_ref[...] + bt_ref[...]


def _lt_ln(x, lt, bn):
    W, b = lt
    g, bt = bn
    n = x.shape[0]
    return pl.pallas_call(
        _mm_ln_body,
        out_shape=jax.ShapeDtypeStruct((n, W.shape[1]), F32),
    )(x, W, b.reshape(1, -1), g.reshape(1, -1), bt.reshape(1, -1))


def _mm_body(x_ref, w_ref, b_ref, o_ref):
    o_ref[...] = jnp.dot(x_ref[...], w_ref[...], preferred_element_type=F32) + b_ref[...]


def _edge_emb_ce_body(ea_ref, w_ref, b_ref, cw_ref, cb_ref, e_out, ce_out):
    e0 = jnp.dot(ea_ref[...], w_ref[...], preferred_element_type=F32) + b_ref[...]
    e_out[...] = e0
    ce_out[...] = jnp.dot(e0, cw_ref[...], preferred_element_type=F32) + cb_ref[...]


def _edge_emb_ce(ea, emb, c1):
    grid = N_EDGES // EDGE_BLK
    k = emb[0].shape[0]
    return pl.pallas_call(
        _edge_emb_ce_body,
        grid=(grid,),
        in_specs=[
            pl.BlockSpec((EDGE_BLK, k), lambda i: (i, 0)),
            pl.BlockSpec((k, EMB), lambda i: (0, 0)),
            pl.BlockSpec((1, EMB), lambda i: (0, 0)),
            pl.BlockSpec((EMB, EMB), lambda i: (0, 0)),
            pl.BlockSpec((1, EMB), lambda i: (0, 0)),
        ],
        out_specs=[
            pl.BlockSpec((EDGE_BLK, EMB), lambda i: (i, 0)),
            pl.BlockSpec((EDGE_BLK, EMB), lambda i: (i, 0)),
        ],
        out_shape=[
            jax.ShapeDtypeStruct((N_EDGES, EMB), F32),
            jax.ShapeDtypeStruct((N_EDGES, EMB), F32),
        ],
    )(ea, emb[0], emb[1].reshape(1, -1), c1[0], c1[1].reshape(1, -1))


def _edge_linear(e, lt):
    W, b = lt
    k, dout = W.shape
    grid = N_EDGES // EDGE_BLK
    return pl.pallas_call(
        _mm_body,
        grid=(grid,),
        in_specs=[
            pl.BlockSpec((EDGE_BLK, k), lambda i: (i, 0)),
            pl.BlockSpec((k, dout), lambda i: (0, 0)),
            pl.BlockSpec((1, dout), lambda i: (0, 0)),
        ],
        out_specs=pl.BlockSpec((EDGE_BLK, dout), lambda i: (i, 0)),
        out_shape=jax.ShapeDtypeStruct((N_EDGES, dout), F32),
    )(e, W, b.reshape(1, -1))


def _node3_body(x_ref, w_ref, b_ref, ax_ref, dx_ref, eb_ref):
    y = jnp.dot(x_ref[...], w_ref[...], preferred_element_type=F32) + b_ref[...]
    ax_ref[...] = y[:, :EMB]
    dx_ref[:, :EMB] = y[:, EMB:2 * EMB]
    dx_ref[:, EMB:] = jnp.zeros((N_NODES, EMB), F32)
    eb_ref[...] = y[:, 2 * EMB:]


def _node3(x, W, b):
    return pl.pallas_call(
        _node3_body,
        out_shape=[
            jax.ShapeDtypeStruct((N_NODES, EMB), F32),
            jax.ShapeDtypeStruct((N_NODES, 2 * EMB), F32),
            jax.ShapeDtypeStruct((N_NODES, 2 * EMB), F32),
        ],
    )(x, W, b.reshape(1, -1))


def _post_full_body(acc_ref, ax_ref, xin_ref, b_ref, g2_ref, b2_ref,
                    w2_ref, bw2_ref, g3_ref, b3_ref, wn_ref, bwn_ref,
                    gn_ref, btn_ref, pool_ref, xn_ref):
    t = acc_ref[0] + acc_ref[1]
    seg0 = t[:PH_ROWS]
    seg1 = t[PH_ROWS:2 * PH_ROWS] - t[:PH_ROWS]
    seg2 = t[2 * PH_ROWS:] - t[PH_ROWS:PH_ROWS + (N_NODES - 2 * PH_ROWS)]
    acc_full = jnp.concatenate([seg0, seg1, seg2], axis=0)
    ssx = acc_full[:, :EMB]
    ss = acc_full[:, EMB:]
    xn = ax_ref[...] + ssx / (ss + 1e-6)
    mu = jnp.mean(xn, axis=0, keepdims=True)
    var = jnp.mean((xn - mu) ** 2, axis=0, keepdims=True)
    xn = jnp.maximum((xn - mu) / jnp.sqrt(var + 1e-5), 0.0)
    x1 = xin_ref[...] + xn
    mu = jnp.mean(x1, axis=1, keepdims=True)
    var = jnp.mean((x1 - mu) ** 2, axis=1, keepdims=True)
    x1 = (x1 - mu) / jnp.sqrt(var + 1e-5) * g2_ref[...] + b2_ref[...]
    y = jnp.dot(x1, w2_ref[...], preferred_element_type=F32) + bw2_ref[...]
    mu = jnp.mean(y, axis=1, keepdims=True)
    var = jnp.mean((y - mu) ** 2, axis=1, keepdims=True)
    xp = (y - mu) / jnp.sqrt(var + 1e-5) * g3_ref[...] + b3_ref[...]

    bv = b_ref[...]
    rows = lax.broadcasted_iota(jnp.int32, (NUM_GRAPHS, 2 * EMB), 0)

    def g_body(g, out):
        m = bv == g
        cnt = jnp.sum(m.astype(F32))
        sm = jnp.sum(jnp.where(m, xp, 0.0), axis=0)
        mx = jnp.max(jnp.where(m, xp, -jnp.inf), axis=0)
        row = jnp.concatenate([mx, sm / jnp.maximum(cnt, 1.0)])
        return jnp.where(rows == g, row[None, :], out)

    pool_ref[...] = lax.fori_loop(0, NUM_GRAPHS, g_body,
                                  jnp.zeros((NUM_GRAPHS, 2 * EMB), F32))

    y = jnp.dot(xp, wn_ref[...], preferred_element_type=F32) + bwn_ref[...]
    mu = jnp.mean(y, axis=1, keepdims=True)
    var = jnp.mean((y - mu) ** 2, axis=1, keepdims=True)
    xn_ref[...] = (y - mu) / jnp.sqrt(var + 1e-5) * gn_ref[...] + btn_ref[...]


def _post_full(acc, ax, xin, batch2d, cp):
    return pl.pallas_call(
        _post_full_body,
        out_shape=[
            jax.ShapeDtypeStruct((NUM_GRAPHS, 2 * EMB), F32),
            jax.ShapeDtypeStruct((N_NODES, EMB), F32),
        ],
    )(acc, ax, xin, batch2d,
      cp['g2'].reshape(1, -1), cp['bt2'].reshape(1, -1),
      cp['w2'], cp['bw2'].reshape(1, -1),
      cp['g3'].reshape(1, -1), cp['bt3'].reshape(1, -1),
      cp['wn'], cp['bwn'].reshape(1, -1),
      cp['gn'].reshape(1, -1), cp['btn'].reshape(1, -1))


def _edge_bn_ce_body(eij_ref, e_ref, st_ref, cw_ref, cb_ref, e_out, ce_out):
    st = st_ref[...]
    s = jnp.sum(st[:, :EMB], axis=0, keepdims=True)
    q = jnp.sum(st[:, EMB:], axis=0, keepdims=True)
    mu = s / N_EDGES
    var = q / N_EDGES - mu * mu
    en = e_ref[...] + jnp.maximum((eij_ref[...] - mu) / jnp.sqrt(var + 1e-5), 0.0)
    e_out[...] = en
    ce_out[...] = jnp.dot(en, cw_ref[...], preferred_element_type=F32) + cb_ref[...]


def _edge_bn_ce(eij, e, stats, cw, cb):
    grid = N_EDGES // EDGE_BLK
    return pl.pallas_call(
        _edge_bn_ce_body,
        grid=(grid,),
        in_specs=[
            pl.BlockSpec((EDGE_BLK, EMB), lambda i: (i, 0)),
            pl.BlockSpec((EDGE_BLK, EMB), lambda i: (i, 0)),
            pl.BlockSpec((NW, 2 * EMB), lambda i: (0, 0)),
            pl.BlockSpec((EMB, EMB), lambda i: (0, 0)),
            pl.BlockSpec((1, EMB), lambda i: (0, 0)),
        ],
        out_specs=[
            pl.BlockSpec((EDGE_BLK, EMB), lambda i: (i, 0)),
            pl.BlockSpec((EDGE_BLK, EMB), lambda i: (i, 0)),
        ],
        out_shape=[
            jax.ShapeDtypeStruct((N_EDGES, EMB), F32),
            jax.ShapeDtypeStruct((N_EDGES, EMB), F32),
        ],
    )(eij, e, stats, cw, cb.reshape(1, -1))


def _classifier_body(x1_ref, x2_ref, x3_ref, w1_ref, b1_ref, g_ref, bt_ref,
                     w2_ref, b2_ref, o_ref):
    h = x1_ref[...] + x2_ref[...] + x3_ref[...]
    h = jnp.maximum(jnp.dot(h, w1_ref[...], preferred_element_type=F32) + b1_ref[...], 0.0)
    mu = jnp.mean(h, axis=1, keepdims=True)
    var = jnp.mean((h - mu) ** 2, axis=1, keepdims=True)
    h = (h - mu) / jnp.sqrt(var + 1e-5) * g_ref[...] + bt_ref[...]
    o_ref[...] = jnp.dot(h, w2_ref[...], preferred_element_type=F32) + b2_ref[...]


def _classifier(x1, x2, x3, c1, bnl, c2):
    return pl.pallas_call(
        _classifier_body,
        out_shape=jax.ShapeDtypeStruct((NUM_GRAPHS, c2[0].shape[1]), F32),
    )(x1, x2, x3, c1[0], c1[1].reshape(1, -1),
      bnl[0].reshape(1, -1), bnl[1].reshape(1, -1),
      c2[0], c2[1].reshape(1, -1))


def _edge_pass(dxT, ebT, ce, src, dst):
    mesh = plsc.VectorSubcoreMesh(core_axis_name="c", subcore_axis_name="s", num_cores=NSC)

    @functools.partial(
        pl.kernel,
        out_type=[
            jax.ShapeDtypeStruct((N_EDGES, EMB), F32),
            jax.ShapeDtypeStruct((NSC, N_NODES, 2 * EMB), F32),
            jax.ShapeDtypeStruct((NW, 1, 2 * EMB), F32),
            jax.ShapeDtypeStruct((N_EDGES, 2 * EMB), F32),
        ],
        scratch_types=[
            pltpu.VMEM((CHUNK,), jnp.int32),
            pltpu.VMEM((CHUNK,), jnp.int32),
            pltpu.VMEM((CHUNK,), jnp.int32),
            pltpu.VMEM((CHUNK, 2 * EMB), F32),
            pltpu.VMEM((CHUNK, 2 * EMB), F32),
            pltpu.VMEM((CHUNK, EMB), F32),
            pltpu.VMEM((CHUNK, EMB), F32),
            pltpu.VMEM((CHUNK, 2 * EMB), F32),
            pltpu.VMEM((1, 2 * EMB), F32),
            pltpu.VMEM((104, 2 * EMB), F32),
            pltpu.VMEM_SHARED((ACC_ROWS, 2 * EMB), F32),
            pltpu.SemaphoreType.DMA,
            pltpu.SemaphoreType.DMA,
        ],
        mesh=mesh,
    )
    def edge_kernel(dx_hbm, eb_hbm, ce_hbm, src_hbm, dst_hbm,
                    eij_o, acc_o, st_o, sv_o,
                    src_v, dst_v, idx_v, dxv, ebv, cev, eijv, scatv, statv,
                    zbuf, acc_sh, sem1, sem2):
        cid = lax.axis_index("c")
        sid = lax.axis_index("s")
        wid = sid * NSC + cid

        zv = jnp.zeros((16,), F32)

        def zb_body(r, carry):
            for c in range(8):
                zbuf[r, pl.ds(c * 16, 16)] = zv
            return carry

        lax.fori_loop(0, 104, zb_body, 0)

        for k in range(2):
            pltpu.sync_copy(zbuf.at[pl.ds(0, 104)],
                            acc_sh.at[pl.ds(sid * ROWS_MAIN + k * 104, 104)])

        @pl.when(sid == 15)
        def _zero_tail():
            pltpu.sync_copy(zbuf.at[pl.ds(0, 16)],
                            acc_sh.at[pl.ds(16 * ROWS_MAIN, 16)])

        def copy_out(node_base, nrows):
            pltpu.sync_copy(acc_sh.at[pl.ds(sid * ROWS_MAIN, ROWS_MAIN)],
                            acc_o.at[cid, pl.ds(node_base + sid * ROWS_MAIN, ROWS_MAIN)])
            tail = nrows - 16 * ROWS_MAIN
            if tail:
                @pl.when(sid == 15)
                def _copy_tail():
                    pltpu.sync_copy(
                        acc_sh.at[pl.ds(16 * ROWS_MAIN, tail)],
                        acc_o.at[cid, pl.ds(node_base + 16 * ROWS_MAIN, tail)])

        plsc.subcore_barrier()

        n_chunks = jnp.where(wid < N_CHUNKS - (N_CHUNKS // NW) * NW,
                             N_CHUNKS // NW + 1, N_CHUNKS // NW)

        def chunk_body(j, st):
            chunk = wid + NW * j
            base = chunk * CHUNK
            pltpu.sync_copy(src_hbm.at[pl.ds(base, CHUNK)], src_v)
            pltpu.sync_copy(dst_hbm.at[pl.ds(base, CHUNK)], dst_v)
            g1 = pltpu.async_copy(dx_hbm.at[dst_v], dxv, sem1)
            g2 = pltpu.async_copy(eb_hbm.at[src_v], ebv, sem2)
            pltpu.sync_copy(ce_hbm.at[pl.ds(base, CHUNK)], cev)
            g1.wait()
            g2.wait()

            def row_body(r, st_in):
                out = list(st_in)
                for c in range(4):
                    sl = pl.ds(c * 16, 16)
                    slb = pl.ds(EMB + c * 16, 16)
                    eij = dxv[r, sl] + ebv[r, sl] + cev[r, sl]
                    sig = 1.0 / (1.0 + jnp.exp(-eij))
                    eijv[r, sl] = eij
                    scatv[r, sl] = sig * ebv[r, slb]
                    scatv[r, slb] = sig
                    out[c] = st_in[c] + eij
                    out[4 + c] = st_in[4 + c] + eij * eij
                return tuple(out)

            st = lax.fori_loop(0, CHUNK, row_body, st)
            for i in range(CHUNK // 16):
                d = dst_v[pl.ds(i * 16, 16)]
                idx_v[pl.ds(i * 16, 16)] = jnp.where(d < PH_ROWS, d, DUMMY)
            pltpu.sync_copy(eijv, eij_o.at[pl.ds(base, CHUNK)])
            pltpu.sync_copy(scatv, sv_o.at[pl.ds(base, CHUNK)])
            pltpu.sync_copy(scatv, acc_sh.at[idx_v], add=True)
            return st

        st0 = tuple(jnp.zeros((16,), F32) for _ in range(8))
        st = lax.fori_loop(0, n_chunks, chunk_body, st0)

        for c in range(4):
            statv[0, pl.ds(c * 16, 16)] = st[c]
            statv[0, pl.ds(EMB + c * 16, 16)] = st[4 + c]
        pltpu.sync_copy(statv, st_o.at[wid])

        plsc.subcore_barrier()
        copy_out(0, PH_ROWS)
        plsc.subcore_barrier()

        for ph in (1, 2):
            lo = ph * PH_ROWS
            hi = min((ph + 1) * PH_ROWS, N_NODES)

            def chunk_body2(j, carry, lo=lo, hi=hi):
                chunk = wid + NW * j
                base = chunk * CHUNK
                pltpu.sync_copy(dst_hbm.at[pl.ds(base, CHUNK)], dst_v)
                pltpu.sync_copy(sv_o.at[pl.ds(base, CHUNK)], scatv)
                for i in range(CHUNK // 16):
                    d = dst_v[pl.ds(i * 16, 16)]
                    idx_v[pl.ds(i * 16, 16)] = jnp.where(
                        (d >= lo) & (d < hi), d - lo, DUMMY)
                pltpu.sync_copy(scatv, acc_sh.at[idx_v], add=True)
                return carry

            lax.fori_loop(0, n_chunks, chunk_body2, 0)
            plsc.subcore_barrier()
            copy_out(lo, hi - lo)
            plsc.subcore_barrier()

    eij, acc, st, _sv = edge_kernel(dxT, ebT, ce, src, dst)
    return eij, acc, st.reshape(NW, 2 * EMB)


def kernel(x, edge_index, edge_attr, batch_index, params):
    p = params
    src = edge_index[0]
    dst = edge_index[1]
    batch2d = batch_index.reshape(N_NODES, 1)

    e0, ce0 = _edge_emb_ce(edge_attr, p['edge_emb'], p['conv1']['C'])
    xx0 = _lt_ln(x, p['lt11'], p['bn11'])

    def stk(*a):
        return jnp.stack(a)

    convs = [p['conv1'], p['conv2'], p['conv3']]
    ps = {
        'w4': stk(*[jnp.concatenate([c['A'][0], c['D'][0], c['E'][0], c['B'][0]], axis=1)
                    for c in convs]),
        'b4': stk(*[jnp.concatenate([c['A'][1], c['D'][1], c['E'][1], c['B'][1]])
                    for c in convs]),
        'cwn': stk(p['conv2']['C'][0], p['conv3']['C'][0], p['conv3']['C'][0]),
        'cbn': stk(p['conv2']['C'][1], p['conv3']['C'][1], p['conv3']['C'][1]),
        'g2': stk(p['bn12'][0], p['bn22'][0], p['bn32'][0]),
        'bt2': stk(p['bn12'][1], p['bn22'][1], p['bn32'][1]),
        'w2': stk(p['lt12'][0], p['lt22'][0], p['lt32'][0]),
        'bw2': stk(p['lt12'][1], p['lt22'][1], p['lt32'][1]),
        'g3': stk(p['bn13'][0], p['bn23'][0], p['bn33'][0]),
        'bt3': stk(p['bn13'][1], p['bn23'][1], p['bn33'][1]),
        'wn': stk(p['lt21'][0], p['lt31'][0], p['lt31'][0]),
        'bwn': stk(p['lt21'][1], p['lt31'][1], p['lt31'][1]),
        'gn': stk(p['bn21'][0], p['bn31'][0], p['bn31'][0]),
        'btn': stk(p['bn21'][1], p['bn31'][1], p['bn31'][1]),
    }

    def conv_step(carry, cp):
        xx, e, ce = carry
        ax, dxT, ebT = _node3(xx, cp['w4'], cp['b4'])
        eij, acc, stats = _edge_pass(dxT, ebT, ce, src, dst)
        pool, xx2 = _post_full(acc, ax, xx, batch2d, cp)
        e2, ce2 = _edge_bn_ce(eij, e, stats, cp['cwn'], cp['cbn'])
        return (xx2, e2, ce2), pool

    _, pools = lax.scan(conv_step, (xx0, e0, ce0), ps)

    return _classifier(pools[0], pools[1], pools[2],
                       p['class1'], p['bnl'], p['class2'])
